# Initial kernel scaffold; baseline (speedup 1.0000x reference)
import jax
import jax.numpy as jnp
from jax.experimental import pallas as pl


def kernel(x, batch):
    raise NotImplementedError("write your pallas kernel here")


# speedup vs baseline: 4.1002x; 4.1002x over previous
import functools

import jax
import jax.numpy as jnp
from jax import lax
from jax.experimental import pallas as pl
from jax.experimental.pallas import tpu as pltpu
from jax.experimental.pallas import tpu_sc as plsc

N_ROWS = 100000
N_COLS = 128
N_SEG = 512
BLK = 80
NBLK = N_ROWS // BLK
N_SUB = 16
N_CORES = 2
N_WORKERS = N_SUB * N_CORES
SEG_PER_SUB = N_SEG // N_SUB
CNT_W = 128


def _sc_body(x_hbm, b_hbm, sums_hbm, cnts_hbm,
             xbuf, idxbuf, onesbuf, zbuf, czbuf, acc_sh, cnt_sh):
    c = lax.axis_index("c")
    sid = lax.axis_index("s")
    wid = sid * N_CORES + c

    zeros16 = jnp.zeros((16,), jnp.float32)
    ones16 = jnp.ones((16,), jnp.float32)
    for i in range(SEG_PER_SUB):
        for j in range(N_COLS // 16):
            zbuf[i, pl.ds(j * 16, 16)] = zeros16
        for j in range(CNT_W // 16):
            czbuf[i, pl.ds(j * 16, 16)] = zeros16
    for i in range(BLK):
        for j in range(CNT_W // 16):
            onesbuf[i, pl.ds(j * 16, 16)] = ones16

    s0 = pl.multiple_of(sid * SEG_PER_SUB, SEG_PER_SUB)
    pltpu.sync_copy(zbuf, acc_sh.at[pl.ds(s0, SEG_PER_SUB)])
    pltpu.sync_copy(czbuf, cnt_sh.at[pl.ds(s0, SEG_PER_SUB)])
    plsc.subcore_barrier()

    max_iters = (NBLK + N_WORKERS - 1) // N_WORKERS

    def blk_body(i, carry):
        b = wid + i * N_WORKERS

        @pl.when(b < NBLK)
        def _():
            r0 = pl.multiple_of(b * BLK, BLK)
            pltpu.sync_copy(b_hbm.at[pl.ds(r0, BLK)], idxbuf)
            pltpu.sync_copy(x_hbm.at[pl.ds(r0, BLK)], xbuf)
            pltpu.sync_copy(xbuf, acc_sh.at[idxbuf], add=True)
            pltpu.sync_copy(onesbuf, cnt_sh.at[idxbuf], add=True)

        return carry

    lax.fori_loop(0, max_iters, blk_body, 0)
    plsc.subcore_barrier()

    pltpu.sync_copy(acc_sh.at[pl.ds(s0, SEG_PER_SUB)], zbuf)
    pltpu.sync_copy(cnt_sh.at[pl.ds(s0, SEG_PER_SUB)], czbuf)
    pltpu.sync_copy(zbuf, sums_hbm.at[c, pl.ds(s0, SEG_PER_SUB)])
    pltpu.sync_copy(czbuf, cnts_hbm.at[c, pl.ds(s0, SEG_PER_SUB)])


@functools.partial(
    pl.kernel,
    mesh=plsc.VectorSubcoreMesh(core_axis_name="c", subcore_axis_name="s"),
    out_type=(
        jax.ShapeDtypeStruct((N_CORES, N_SEG, N_COLS), jnp.float32),
        jax.ShapeDtypeStruct((N_CORES, N_SEG, CNT_W), jnp.float32),
    ),
    scratch_types=[
        pltpu.VMEM((BLK, N_COLS), jnp.float32),
        pltpu.VMEM((BLK,), jnp.int32),
        pltpu.VMEM((BLK, CNT_W), jnp.float32),
        pltpu.VMEM((SEG_PER_SUB, N_COLS), jnp.float32),
        pltpu.VMEM((SEG_PER_SUB, CNT_W), jnp.float32),
        pltpu.VMEM_SHARED((N_SEG, N_COLS), jnp.float32),
        pltpu.VMEM_SHARED((N_SEG, CNT_W), jnp.float32),
    ],
)
def _sc_pool(x_hbm, b_hbm, sums_hbm, cnts_hbm, *scratch):
    _sc_body(x_hbm, b_hbm, sums_hbm, cnts_hbm, *scratch)


def _combine_body(p_ref, c_ref, o_ref):
    sums = p_ref[0] + p_ref[1]
    cnt = c_ref[0] + c_ref[1]
    cnt1 = jnp.maximum(cnt[:, 0:1], 1.0)
    o_ref[...] = sums / cnt1


_combine = pl.pallas_call(
    _combine_body,
    out_shape=jax.ShapeDtypeStruct((N_SEG, N_COLS), jnp.float32),
)


def kernel(x, batch):
    sums_p, cnts_p = _sc_pool(x, batch.astype(jnp.int32))
    return _combine(sums_p, cnts_p)


# speedup vs baseline: 6.7800x; 1.6536x over previous
import functools

import jax
import jax.numpy as jnp
from jax import lax
from jax.experimental import pallas as pl
from jax.experimental.pallas import tpu as pltpu
from jax.experimental.pallas import tpu_sc as plsc

N_ROWS = 100000
N_COLS = 128
N_SEG = 512
BLK = 80
NBLK = N_ROWS // BLK
N_SUB = 16
N_CORES = 2
N_WORKERS = N_SUB * N_CORES
SEG_PER_SUB = N_SEG // N_SUB
CNT_W = 128


def _sc_body(x_hbm, b_hbm, sums_hbm, cnts_hbm,
             xb0, xb1, ib0, ib1, onesbuf, zbuf, czbuf, acc_sh, cnt_sh,
             semi0, semx0, semi1, semx1, semsc):
    c = lax.axis_index("c")
    sid = lax.axis_index("s")
    wid = sid * N_CORES + c

    def start_load(ib, xb, semi, semx, b):
        r0 = pl.multiple_of(b * BLK, BLK)
        pltpu.async_copy(b_hbm.at[pl.ds(r0, BLK)], ib, semi)
        pltpu.async_copy(x_hbm.at[pl.ds(r0, BLK)], xb, semx)

    def wait_load(ib, xb, semi, semx, b):
        r0 = pl.multiple_of(b * BLK, BLK)
        pltpu.make_async_copy(b_hbm.at[pl.ds(r0, BLK)], ib, semi).wait()
        pltpu.make_async_copy(x_hbm.at[pl.ds(r0, BLK)], xb, semx).wait()

    def scatter_block(ib, xb):
        pltpu.async_copy(xb, acc_sh.at[ib], semsc, add=True)
        pltpu.async_copy(onesbuf, cnt_sh.at[ib], semsc, add=True)
        pltpu.make_async_copy(xb, acc_sh.at[ib], semsc).wait()
        pltpu.make_async_copy(onesbuf, cnt_sh.at[ib], semsc).wait()

    zeros16 = jnp.zeros((16,), jnp.float32)
    ones16 = jnp.ones((16,), jnp.float32)
    for i in range(SEG_PER_SUB):
        for j in range(N_COLS // 16):
            zbuf[i, pl.ds(j * 16, 16)] = zeros16
        for j in range(CNT_W // 16):
            czbuf[i, pl.ds(j * 16, 16)] = zeros16
    for i in range(BLK):
        for j in range(CNT_W // 16):
            onesbuf[i, pl.ds(j * 16, 16)] = ones16

    s0 = pl.multiple_of(sid * SEG_PER_SUB, SEG_PER_SUB)
    pltpu.sync_copy(zbuf, acc_sh.at[pl.ds(s0, SEG_PER_SUB)])
    pltpu.sync_copy(czbuf, cnt_sh.at[pl.ds(s0, SEG_PER_SUB)])
    plsc.subcore_barrier()

    nblk_w = 39 + jnp.where(wid < NBLK - 39 * N_WORKERS, 1, 0)

    start_load(ib0, xb0, semi0, semx0, wid)

    def pair_body(i, carry):
        j0 = 2 * i
        j1 = 2 * i + 1
        j2 = 2 * i + 2
        b0 = wid + j0 * N_WORKERS
        b1 = wid + j1 * N_WORKERS
        b2 = wid + j2 * N_WORKERS

        wait_load(ib0, xb0, semi0, semx0, b0)

        @pl.when(j1 < nblk_w)
        def _():
            start_load(ib1, xb1, semi1, semx1, b1)

        scatter_block(ib0, xb0)

        @pl.when(j1 < nblk_w)
        def _():
            wait_load(ib1, xb1, semi1, semx1, b1)

            @pl.when(j2 < nblk_w)
            def _():
                start_load(ib0, xb0, semi0, semx0, b2)

            scatter_block(ib1, xb1)

        return carry

    lax.fori_loop(0, 20, pair_body, 0)
    plsc.subcore_barrier()

    pltpu.sync_copy(acc_sh.at[pl.ds(s0, SEG_PER_SUB)], zbuf)
    pltpu.sync_copy(cnt_sh.at[pl.ds(s0, SEG_PER_SUB)], czbuf)
    pltpu.sync_copy(zbuf, sums_hbm.at[c, pl.ds(s0, SEG_PER_SUB)])
    pltpu.sync_copy(czbuf, cnts_hbm.at[c, pl.ds(s0, SEG_PER_SUB)])


@functools.partial(
    pl.kernel,
    mesh=plsc.VectorSubcoreMesh(core_axis_name="c", subcore_axis_name="s"),
    out_type=(
        jax.ShapeDtypeStruct((N_CORES, N_SEG, N_COLS), jnp.float32),
        jax.ShapeDtypeStruct((N_CORES, N_SEG, CNT_W), jnp.float32),
    ),
    scratch_types=[
        pltpu.VMEM((BLK, N_COLS), jnp.float32),
        pltpu.VMEM((BLK, N_COLS), jnp.float32),
        pltpu.VMEM((BLK,), jnp.int32),
        pltpu.VMEM((BLK,), jnp.int32),
        pltpu.VMEM((BLK, CNT_W), jnp.float32),
        pltpu.VMEM((SEG_PER_SUB, N_COLS), jnp.float32),
        pltpu.VMEM((SEG_PER_SUB, CNT_W), jnp.float32),
        pltpu.VMEM_SHARED((N_SEG, N_COLS), jnp.float32),
        pltpu.VMEM_SHARED((N_SEG, CNT_W), jnp.float32),
        pltpu.SemaphoreType.DMA,
        pltpu.SemaphoreType.DMA,
        pltpu.SemaphoreType.DMA,
        pltpu.SemaphoreType.DMA,
        pltpu.SemaphoreType.DMA,
    ],
)
def _sc_pool(x_hbm, b_hbm, sums_hbm, cnts_hbm, *scratch):
    _sc_body(x_hbm, b_hbm, sums_hbm, cnts_hbm, *scratch)


def _combine_body(p_ref, c_ref, o_ref):
    sums = p_ref[0] + p_ref[1]
    cnt = c_ref[0] + c_ref[1]
    cnt1 = jnp.maximum(cnt[:, 0:1], 1.0)
    o_ref[...] = sums / cnt1


_combine = pl.pallas_call(
    _combine_body,
    out_shape=jax.ShapeDtypeStruct((N_SEG, N_COLS), jnp.float32),
)


def kernel(x, batch):
    sums_p, cnts_p = _sc_pool(x, batch.astype(jnp.int32))
    return _combine(sums_p, cnts_p)
